# baseline (speedup 1.0000x reference)
import functools

import jax
import jax.numpy as jnp
from jax import lax
from jax.experimental import pallas as pl
from jax.experimental.pallas import tpu as pltpu
from jax.experimental.pallas import tpu_sc as plsc

NC = 2
NS = 16
NW = NC * NS
G = 64


def _sc_aggregate(h, ei_r, zeros_nd, *, n, d, chunks, k, groups):
    sg = chunks // groups
    rpt = (n // NS) & ~7
    last_rows = n - (NS - 1) * rpt
    mesh = plsc.VectorSubcoreMesh(core_axis_name="c", subcore_axis_name="s",
                                  num_cores=NC, num_subcores=NS)

    @functools.partial(
        pl.kernel,
        out_type=jax.ShapeDtypeStruct((NC, n, d), jnp.float32),
        mesh=mesh,
        scratch_types=[
            pltpu.VMEM((sg, k), jnp.int32),
            pltpu.VMEM((sg, k), jnp.int32),
            pltpu.VMEM((k, d), jnp.float32),
            pltpu.VMEM((k, d), jnp.float32),
            pltpu.VMEM_SHARED((n, d), jnp.float32),
            pltpu.SemaphoreType.DMA,
            pltpu.SemaphoreType.DMA,
            pltpu.SemaphoreType.DMA,
            pltpu.SemaphoreType.DMA,
        ],
    )
    def agg_kernel(h_hbm, ei_hbm, z_hbm, out_hbm,
                   src_v, dst_v, rows0_v, rows1_v, agg_sh,
                   sem0, sem1, ssem0, ssem1):
        c = lax.axis_index("c")
        s = lax.axis_index("s")
        wid = s * NC + c
        row0 = pl.multiple_of(s * rpt, 8)

        def each_slice(fn):
            @pl.when(s < NS - 1)
            def _():
                fn(row0, rpt)

            @pl.when(s == NS - 1)
            def _():
                fn((NS - 1) * rpt, last_rows)

        each_slice(lambda r0, nr: pltpu.sync_copy(
            z_hbm.at[pl.ds(r0, nr)], agg_sh.at[pl.ds(r0, nr)]))
        plsc.subcore_barrier()

        for g in range(groups):
            pltpu.sync_copy(ei_hbm.at[0, wid, g], src_v)
            pltpu.sync_copy(ei_hbm.at[1, wid, g], dst_v)
            pltpu.async_copy(h_hbm.at[src_v.at[0]], rows0_v, sem0)
            pltpu.async_copy(h_hbm.at[src_v.at[1]], rows1_v, sem1)

            def body(j, carry):
                pltpu.make_async_copy(
                    h_hbm.at[src_v.at[j]], rows0_v, sem0).wait()
                pltpu.async_copy(rows0_v, agg_sh.at[dst_v.at[j]], ssem0,
                                 add=True)
                pltpu.make_async_copy(
                    h_hbm.at[src_v.at[j + 1]], rows1_v, sem1).wait()
                pltpu.async_copy(rows1_v, agg_sh.at[dst_v.at[j + 1]], ssem1,
                                 add=True)
                pltpu.make_async_copy(
                    rows0_v, agg_sh.at[dst_v.at[j]], ssem0).wait()

                @pl.when(j + 2 < sg)
                def _():
                    pltpu.async_copy(h_hbm.at[src_v.at[j + 2]], rows0_v, sem0)

                pltpu.make_async_copy(
                    rows1_v, agg_sh.at[dst_v.at[j + 1]], ssem1).wait()

                @pl.when(j + 3 < sg)
                def _():
                    pltpu.async_copy(h_hbm.at[src_v.at[j + 3]], rows1_v, sem1)

                return carry

            lax.fori_loop(0, sg // 2, lambda i, c: body(i * 2, c), 0)
        plsc.subcore_barrier()
        each_slice(lambda r0, nr: pltpu.sync_copy(
            agg_sh.at[pl.ds(r0, nr)], out_hbm.at[c, pl.ds(r0, nr)]))

    return agg_kernel(h, ei_r, zeros_nd)


def _tc_layer(aggp, h, wr, br_r, wroot, *, n, d, bn):
    nb = n // bn

    def body(ap, xb, wr_ref, b_ref, wroot_ref, o):
        agg = ap[0] + ap[1]
        o[...] = jnp.maximum(
            jnp.dot(agg, wr_ref[...], preferred_element_type=jnp.float32)
            + b_ref[...]
            + jnp.dot(xb[...], wroot_ref[...],
                      preferred_element_type=jnp.float32),
            0.0)

    return pl.pallas_call(
        body,
        grid=(nb,),
        in_specs=[
            pl.BlockSpec((NC, bn, d), lambda i: (0, i, 0)),
            pl.BlockSpec((bn, d), lambda i: (i, 0)),
            pl.BlockSpec((d, d), lambda i: (0, 0)),
            pl.BlockSpec((1, d), lambda i: (0, 0)),
            pl.BlockSpec((d, d), lambda i: (0, 0)),
        ],
        out_specs=pl.BlockSpec((bn, d), lambda i: (i, 0)),
        out_shape=jax.ShapeDtypeStruct((n, d), jnp.float32),
    )(aggp, h, wr, br_r, wroot)


def _tc_final(aggp, h, batch_r, wr, br_r, wroot, w1, b1_r, w2, b2_r,
              *, n, d, c_out, bn):
    nb = n // bn

    def body(ap, xb, bt, wr_ref, b_ref, wroot_ref,
             w1_ref, b1_ref, w2_ref, b2_ref, emb, out):
        i = pl.program_id(0)
        h3 = jnp.maximum(
            jnp.dot(ap[0] + ap[1], wr_ref[...],
                    preferred_element_type=jnp.float32)
            + b_ref[...]
            + jnp.dot(xb[...], wroot_ref[...],
                      preferred_element_type=jnp.float32),
            0.0)
        bvec = bt[0, 0, :]
        gids = lax.broadcasted_iota(jnp.int32, (G, bn), 0)
        mask = (gids == bvec[None, :]).astype(jnp.float32)
        part = jnp.dot(mask, h3, preferred_element_type=jnp.float32)

        @pl.when(i == 0)
        def _():
            emb[...] = part

        @pl.when(i > 0)
        def _():
            emb[...] = emb[...] + part

        @pl.when(i == nb - 1)
        def _():
            y = jnp.maximum(
                jnp.dot(emb[...], w1_ref[...],
                        preferred_element_type=jnp.float32) + b1_ref[...],
                0.0)
            out[...] = (jnp.dot(y, w2_ref[...],
                                preferred_element_type=jnp.float32)
                        + b2_ref[...])

    emb, out = pl.pallas_call(
        body,
        grid=(nb,),
        in_specs=[
            pl.BlockSpec((NC, bn, d), lambda i: (0, i, 0)),
            pl.BlockSpec((bn, d), lambda i: (i, 0)),
            pl.BlockSpec((1, 1, bn), lambda i: (i, 0, 0)),
            pl.BlockSpec((d, d), lambda i: (0, 0)),
            pl.BlockSpec((1, d), lambda i: (0, 0)),
            pl.BlockSpec((d, d), lambda i: (0, 0)),
            pl.BlockSpec((d, d), lambda i: (0, 0)),
            pl.BlockSpec((1, d), lambda i: (0, 0)),
            pl.BlockSpec((d, c_out), lambda i: (0, 0)),
            pl.BlockSpec((1, c_out), lambda i: (0, 0)),
        ],
        out_specs=[
            pl.BlockSpec((G, d), lambda i: (0, 0)),
            pl.BlockSpec((G, c_out), lambda i: (0, 0)),
        ],
        out_shape=[
            jax.ShapeDtypeStruct((G, d), jnp.float32),
            jax.ShapeDtypeStruct((G, c_out), jnp.float32),
        ],
    )(aggp, h, batch_r, wr, br_r, wroot, w1, b1_r, w2, b2_r)
    return emb, out


def kernel(x, edge_index, batch, Wr1, br1, Wroot1, Wr2, br2, Wroot2,
           Wr3, br3, Wroot3, W1, b1, W2, b2):
    n, d = x.shape
    e = edge_index.shape[1]
    c_out = W2.shape[1]

    k = 125
    groups = 2
    per_w = e // NW
    chunks = per_w // k
    ei_r = edge_index.reshape(2, NW, groups, chunks // groups, k)
    zeros_nd = jnp.zeros((n, d), jnp.float32)

    bn = 2000
    batch_r = batch.reshape(n // bn, 1, bn)
    br1_r = br1.reshape(1, d)
    br2_r = br2.reshape(1, d)
    br3_r = br3.reshape(1, d)
    b1_r = b1.reshape(1, d)
    b2_r = b2.reshape(1, c_out)

    sc_args = dict(n=n, d=d, chunks=chunks, k=k, groups=groups)
    aggp = _sc_aggregate(x, ei_r, zeros_nd, **sc_args)
    h1 = _tc_layer(aggp, x, Wr1, br1_r, Wroot1, n=n, d=d, bn=bn)
    aggp = _sc_aggregate(h1, ei_r, zeros_nd, **sc_args)
    h2 = _tc_layer(aggp, h1, Wr2, br2_r, Wroot2, n=n, d=d, bn=bn)
    aggp = _sc_aggregate(h2, ei_r, zeros_nd, **sc_args)
    emb, out = _tc_final(aggp, h2, batch_r, Wr3, br3_r, Wroot3,
                         W1, b1_r, W2, b2_r, n=n, d=d, c_out=c_out, bn=bn)
    return (out, emb)


# speedup vs baseline: 1.2833x; 1.2833x over previous
import functools

import jax
import jax.numpy as jnp
from jax import lax
from jax.experimental import pallas as pl
from jax.experimental.pallas import tpu as pltpu
from jax.experimental.pallas import tpu_sc as plsc

NC = 2
NS = 16
NW = NC * NS
G = 64


def _sc_aggregate(h, ei_r, zeros_nd, *, n, d, chunks, k, groups):
    sg = chunks // groups
    rpt = (n // NS) & ~7
    last_rows = n - (NS - 1) * rpt
    mesh = plsc.VectorSubcoreMesh(core_axis_name="c", subcore_axis_name="s",
                                  num_cores=NC, num_subcores=NS)

    @functools.partial(
        pl.kernel,
        out_type=jax.ShapeDtypeStruct((NC, n, d), jnp.float32),
        mesh=mesh,
        scratch_types=[
            pltpu.VMEM((sg, k), jnp.int32),
            pltpu.VMEM((sg, k), jnp.int32),
            pltpu.VMEM((k, d), jnp.float32),
            pltpu.VMEM((k, d), jnp.float32),
            pltpu.VMEM_SHARED((n, d), jnp.float32),
            pltpu.SemaphoreType.DMA,
            pltpu.SemaphoreType.DMA,
        ],
    )
    def agg_kernel(h_hbm, ei_hbm, z_hbm, out_hbm,
                   src_v, dst_v, rows0_v, rows1_v, agg_sh, sem0, sem1):
        c = lax.axis_index("c")
        s = lax.axis_index("s")
        wid = s * NC + c
        row0 = pl.multiple_of(s * rpt, 8)

        def each_slice(fn):
            @pl.when(s < NS - 1)
            def _():
                fn(row0, rpt)

            @pl.when(s == NS - 1)
            def _():
                fn((NS - 1) * rpt, last_rows)

        each_slice(lambda r0, nr: pltpu.sync_copy(
            z_hbm.at[pl.ds(r0, nr)], agg_sh.at[pl.ds(r0, nr)]))
        plsc.subcore_barrier()

        for g in range(groups):
            pltpu.sync_copy(ei_hbm.at[0, wid, g], src_v)
            pltpu.sync_copy(ei_hbm.at[1, wid, g], dst_v)
            pltpu.async_copy(h_hbm.at[src_v.at[0]], rows0_v, sem0)

            def body(j, carry):
                pltpu.async_copy(h_hbm.at[src_v.at[j + 1]], rows1_v, sem1)
                pltpu.make_async_copy(
                    h_hbm.at[src_v.at[j]], rows0_v, sem0).wait()
                pltpu.sync_copy(rows0_v, agg_sh.at[dst_v.at[j]], add=True)

                @pl.when(j + 2 < sg)
                def _():
                    pltpu.async_copy(h_hbm.at[src_v.at[j + 2]], rows0_v, sem0)

                pltpu.make_async_copy(
                    h_hbm.at[src_v.at[j + 1]], rows1_v, sem1).wait()
                pltpu.sync_copy(rows1_v, agg_sh.at[dst_v.at[j + 1]], add=True)
                return carry

            lax.fori_loop(0, sg // 2, lambda i, c: body(i * 2, c), 0)
        plsc.subcore_barrier()
        each_slice(lambda r0, nr: pltpu.sync_copy(
            agg_sh.at[pl.ds(r0, nr)], out_hbm.at[c, pl.ds(r0, nr)]))

    return agg_kernel(h, ei_r, zeros_nd)


def _tc_layer(aggp, h, wr, br_r, wroot, *, n, d, bn):
    nb = n // bn

    def body(ap, xb, wr_ref, b_ref, wroot_ref, o):
        agg = ap[0] + ap[1]
        o[...] = jnp.maximum(
            jnp.dot(agg, wr_ref[...], preferred_element_type=jnp.float32)
            + b_ref[...]
            + jnp.dot(xb[...], wroot_ref[...],
                      preferred_element_type=jnp.float32),
            0.0)

    return pl.pallas_call(
        body,
        grid=(nb,),
        in_specs=[
            pl.BlockSpec((NC, bn, d), lambda i: (0, i, 0)),
            pl.BlockSpec((bn, d), lambda i: (i, 0)),
            pl.BlockSpec((d, d), lambda i: (0, 0)),
            pl.BlockSpec((1, d), lambda i: (0, 0)),
            pl.BlockSpec((d, d), lambda i: (0, 0)),
        ],
        out_specs=pl.BlockSpec((bn, d), lambda i: (i, 0)),
        out_shape=jax.ShapeDtypeStruct((n, d), jnp.float32),
    )(aggp, h, wr, br_r, wroot)


def _tc_final(aggp, h, batch_r, wr, br_r, wroot, w1, b1_r, w2, b2_r,
              *, n, d, c_out, bn):
    nb = n // bn

    def body(ap, xb, bt, wr_ref, b_ref, wroot_ref,
             w1_ref, b1_ref, w2_ref, b2_ref, emb, out):
        i = pl.program_id(0)
        h3 = jnp.maximum(
            jnp.dot(ap[0] + ap[1], wr_ref[...],
                    preferred_element_type=jnp.float32)
            + b_ref[...]
            + jnp.dot(xb[...], wroot_ref[...],
                      preferred_element_type=jnp.float32),
            0.0)
        bvec = bt[0, 0, :]
        gids = lax.broadcasted_iota(jnp.int32, (G, bn), 0)
        mask = (gids == bvec[None, :]).astype(jnp.float32)
        part = jnp.dot(mask, h3, preferred_element_type=jnp.float32)

        @pl.when(i == 0)
        def _():
            emb[...] = part

        @pl.when(i > 0)
        def _():
            emb[...] = emb[...] + part

        @pl.when(i == nb - 1)
        def _():
            y = jnp.maximum(
                jnp.dot(emb[...], w1_ref[...],
                        preferred_element_type=jnp.float32) + b1_ref[...],
                0.0)
            out[...] = (jnp.dot(y, w2_ref[...],
                                preferred_element_type=jnp.float32)
                        + b2_ref[...])

    emb, out = pl.pallas_call(
        body,
        grid=(nb,),
        in_specs=[
            pl.BlockSpec((NC, bn, d), lambda i: (0, i, 0)),
            pl.BlockSpec((bn, d), lambda i: (i, 0)),
            pl.BlockSpec((1, 1, bn), lambda i: (i, 0, 0)),
            pl.BlockSpec((d, d), lambda i: (0, 0)),
            pl.BlockSpec((1, d), lambda i: (0, 0)),
            pl.BlockSpec((d, d), lambda i: (0, 0)),
            pl.BlockSpec((d, d), lambda i: (0, 0)),
            pl.BlockSpec((1, d), lambda i: (0, 0)),
            pl.BlockSpec((d, c_out), lambda i: (0, 0)),
            pl.BlockSpec((1, c_out), lambda i: (0, 0)),
        ],
        out_specs=[
            pl.BlockSpec((G, d), lambda i: (0, 0)),
            pl.BlockSpec((G, c_out), lambda i: (0, 0)),
        ],
        out_shape=[
            jax.ShapeDtypeStruct((G, d), jnp.float32),
            jax.ShapeDtypeStruct((G, c_out), jnp.float32),
        ],
    )(aggp, h, batch_r, wr, br_r, wroot, w1, b1_r, w2, b2_r)
    return emb, out


def kernel(x, edge_index, batch, Wr1, br1, Wroot1, Wr2, br2, Wroot2,
           Wr3, br3, Wroot3, W1, b1, W2, b2):
    n, d = x.shape
    e = edge_index.shape[1]
    c_out = W2.shape[1]

    k = 125
    groups = 2
    per_w = e // NW
    chunks = per_w // k
    ei_r = edge_index.reshape(2, NW, groups, chunks // groups, k)
    zeros_nd = jnp.zeros((n, d), jnp.float32)

    bn = 2000
    batch_r = batch.reshape(n // bn, 1, bn)
    br1_r = br1.reshape(1, d)
    br2_r = br2.reshape(1, d)
    br3_r = br3.reshape(1, d)
    b1_r = b1.reshape(1, d)
    b2_r = b2.reshape(1, c_out)

    sc_args = dict(n=n, d=d, chunks=chunks, k=k, groups=groups)
    aggp = _sc_aggregate(x, ei_r, zeros_nd, **sc_args)
    h1 = _tc_layer(aggp, x, Wr1, br1_r, Wroot1, n=n, d=d, bn=bn)
    aggp = _sc_aggregate(h1, ei_r, zeros_nd, **sc_args)
    h2 = _tc_layer(aggp, h1, Wr2, br2_r, Wroot2, n=n, d=d, bn=bn)
    aggp = _sc_aggregate(h2, ei_r, zeros_nd, **sc_args)
    emb, out = _tc_final(aggp, h2, batch_r, Wr3, br3_r, Wroot3,
                         W1, b1_r, W2, b2_r, n=n, d=d, c_out=c_out, bn=bn)
    return (out, emb)


# speedup vs baseline: 1.2969x; 1.0107x over previous
import functools

import jax
import jax.numpy as jnp
from jax import lax
from jax.experimental import pallas as pl
from jax.experimental.pallas import tpu as pltpu
from jax.experimental.pallas import tpu_sc as plsc

NC = 2
NS = 16
NW = NC * NS
G = 64


def _sc_aggregate(h, ei_r, zeros_nd, *, n, d, chunks, k, groups):
    sg = chunks // groups
    rpt = (n // NS) & ~7
    last_rows = n - (NS - 1) * rpt
    mesh = plsc.VectorSubcoreMesh(core_axis_name="c", subcore_axis_name="s",
                                  num_cores=NC, num_subcores=NS)

    @functools.partial(
        pl.kernel,
        out_type=jax.ShapeDtypeStruct((NC, n, d), jnp.float32),
        mesh=mesh,
        scratch_types=[
            pltpu.VMEM((sg, k), jnp.int32),
            pltpu.VMEM((sg, k), jnp.int32),
            pltpu.VMEM((k, d), jnp.float32),
            pltpu.VMEM((k, d), jnp.float32),
            pltpu.VMEM_SHARED((n, d), jnp.float32),
            pltpu.SemaphoreType.DMA,
            pltpu.SemaphoreType.DMA,
            pltpu.SemaphoreType.DMA,
        ],
    )
    def agg_kernel(h_hbm, ei_hbm, z_hbm, out_hbm,
                   src_v, dst_v, rows0_v, rows1_v, agg_sh, sem0, sem1, semz):
        c = lax.axis_index("c")
        s = lax.axis_index("s")
        wid = s * NC + c
        row0 = pl.multiple_of(s * rpt, 8)

        def each_slice(fn):
            @pl.when(s < NS - 1)
            def _():
                fn(row0, rpt)

            @pl.when(s == NS - 1)
            def _():
                fn((NS - 1) * rpt, last_rows)

        each_slice(lambda r0, nr: pltpu.async_copy(
            z_hbm.at[pl.ds(r0, nr)], agg_sh.at[pl.ds(r0, nr)], semz))

        for g in range(groups):
            if g == 0:
                pltpu.async_copy(ei_hbm.at[0, wid, 0], src_v, sem0)
                pltpu.async_copy(ei_hbm.at[1, wid, 0], dst_v, sem1)
                pltpu.make_async_copy(ei_hbm.at[0, wid, 0], src_v, sem0).wait()
                pltpu.make_async_copy(ei_hbm.at[1, wid, 0], dst_v, sem1).wait()
                each_slice(lambda r0, nr: pltpu.make_async_copy(
                    z_hbm.at[pl.ds(r0, nr)], agg_sh.at[pl.ds(r0, nr)],
                    semz).wait())
                plsc.subcore_barrier()
            else:
                pltpu.sync_copy(ei_hbm.at[0, wid, g], src_v)
                pltpu.sync_copy(ei_hbm.at[1, wid, g], dst_v)
            pltpu.async_copy(h_hbm.at[src_v.at[0]], rows0_v, sem0)

            def body(j, carry):
                pltpu.async_copy(h_hbm.at[src_v.at[j + 1]], rows1_v, sem1)
                pltpu.make_async_copy(
                    h_hbm.at[src_v.at[j]], rows0_v, sem0).wait()
                pltpu.sync_copy(rows0_v, agg_sh.at[dst_v.at[j]], add=True)

                @pl.when(j + 2 < sg)
                def _():
                    pltpu.async_copy(h_hbm.at[src_v.at[j + 2]], rows0_v, sem0)

                pltpu.make_async_copy(
                    h_hbm.at[src_v.at[j + 1]], rows1_v, sem1).wait()
                pltpu.sync_copy(rows1_v, agg_sh.at[dst_v.at[j + 1]], add=True)
                return carry

            lax.fori_loop(0, sg // 2, lambda i, c: body(i * 2, c), 0)
        plsc.subcore_barrier()
        each_slice(lambda r0, nr: pltpu.sync_copy(
            agg_sh.at[pl.ds(r0, nr)], out_hbm.at[c, pl.ds(r0, nr)]))

    return agg_kernel(h, ei_r, zeros_nd)


def _tc_layer(aggp, h, wr, br_r, wroot, *, n, d, bn):
    nb = n // bn

    def body(ap, xb, wr_ref, b_ref, wroot_ref, o):
        agg = ap[0] + ap[1]
        o[...] = jnp.maximum(
            jnp.dot(agg, wr_ref[...], preferred_element_type=jnp.float32)
            + b_ref[...]
            + jnp.dot(xb[...], wroot_ref[...],
                      preferred_element_type=jnp.float32),
            0.0)

    return pl.pallas_call(
        body,
        grid=(nb,),
        in_specs=[
            pl.BlockSpec((NC, bn, d), lambda i: (0, i, 0)),
            pl.BlockSpec((bn, d), lambda i: (i, 0)),
            pl.BlockSpec((d, d), lambda i: (0, 0)),
            pl.BlockSpec((1, d), lambda i: (0, 0)),
            pl.BlockSpec((d, d), lambda i: (0, 0)),
        ],
        out_specs=pl.BlockSpec((bn, d), lambda i: (i, 0)),
        out_shape=jax.ShapeDtypeStruct((n, d), jnp.float32),
    )(aggp, h, wr, br_r, wroot)


def _tc_final(aggp, h, batch_r, wr, br_r, wroot, w1, b1_r, w2, b2_r,
              *, n, d, c_out, bn):
    nb = n // bn

    def body(ap, xb, bt, wr_ref, b_ref, wroot_ref,
             w1_ref, b1_ref, w2_ref, b2_ref, emb, out):
        i = pl.program_id(0)
        h3 = jnp.maximum(
            jnp.dot(ap[0] + ap[1], wr_ref[...],
                    preferred_element_type=jnp.float32)
            + b_ref[...]
            + jnp.dot(xb[...], wroot_ref[...],
                      preferred_element_type=jnp.float32),
            0.0)
        bvec = bt[0, 0, :]
        gids = lax.broadcasted_iota(jnp.int32, (G, bn), 0)
        mask = (gids == bvec[None, :]).astype(jnp.float32)
        part = jnp.dot(mask, h3, preferred_element_type=jnp.float32)

        @pl.when(i == 0)
        def _():
            emb[...] = part

        @pl.when(i > 0)
        def _():
            emb[...] = emb[...] + part

        @pl.when(i == nb - 1)
        def _():
            y = jnp.maximum(
                jnp.dot(emb[...], w1_ref[...],
                        preferred_element_type=jnp.float32) + b1_ref[...],
                0.0)
            out[...] = (jnp.dot(y, w2_ref[...],
                                preferred_element_type=jnp.float32)
                        + b2_ref[...])

    emb, out = pl.pallas_call(
        body,
        grid=(nb,),
        in_specs=[
            pl.BlockSpec((NC, bn, d), lambda i: (0, i, 0)),
            pl.BlockSpec((bn, d), lambda i: (i, 0)),
            pl.BlockSpec((1, 1, bn), lambda i: (i, 0, 0)),
            pl.BlockSpec((d, d), lambda i: (0, 0)),
            pl.BlockSpec((1, d), lambda i: (0, 0)),
            pl.BlockSpec((d, d), lambda i: (0, 0)),
            pl.BlockSpec((d, d), lambda i: (0, 0)),
            pl.BlockSpec((1, d), lambda i: (0, 0)),
            pl.BlockSpec((d, c_out), lambda i: (0, 0)),
            pl.BlockSpec((1, c_out), lambda i: (0, 0)),
        ],
        out_specs=[
            pl.BlockSpec((G, d), lambda i: (0, 0)),
            pl.BlockSpec((G, c_out), lambda i: (0, 0)),
        ],
        out_shape=[
            jax.ShapeDtypeStruct((G, d), jnp.float32),
            jax.ShapeDtypeStruct((G, c_out), jnp.float32),
        ],
    )(aggp, h, batch_r, wr, br_r, wroot, w1, b1_r, w2, b2_r)
    return emb, out


def kernel(x, edge_index, batch, Wr1, br1, Wroot1, Wr2, br2, Wroot2,
           Wr3, br3, Wroot3, W1, b1, W2, b2):
    n, d = x.shape
    e = edge_index.shape[1]
    c_out = W2.shape[1]

    k = 125
    groups = 2
    per_w = e // NW
    chunks = per_w // k
    ei_r = edge_index.reshape(2, NW, groups, chunks // groups, k)
    zeros_nd = jnp.zeros((n, d), jnp.float32)

    bn = 2000
    batch_r = batch.reshape(n // bn, 1, bn)
    br1_r = br1.reshape(1, d)
    br2_r = br2.reshape(1, d)
    br3_r = br3.reshape(1, d)
    b1_r = b1.reshape(1, d)
    b2_r = b2.reshape(1, c_out)

    sc_args = dict(n=n, d=d, chunks=chunks, k=k, groups=groups)
    aggp = _sc_aggregate(x, ei_r, zeros_nd, **sc_args)
    h1 = _tc_layer(aggp, x, Wr1, br1_r, Wroot1, n=n, d=d, bn=bn)
    aggp = _sc_aggregate(h1, ei_r, zeros_nd, **sc_args)
    h2 = _tc_layer(aggp, h1, Wr2, br2_r, Wroot2, n=n, d=d, bn=bn)
    aggp = _sc_aggregate(h2, ei_r, zeros_nd, **sc_args)
    emb, out = _tc_final(aggp, h2, batch_r, Wr3, br3_r, Wroot3,
                         W1, b1_r, W2, b2_r, n=n, d=d, c_out=c_out, bn=bn)
    return (out, emb)
